# Initial kernel scaffold; baseline (speedup 1.0000x reference)
import jax
import jax.numpy as jnp
from jax.experimental import pallas as pl


def kernel(x_user, x_pc, x_url, ei_pc, ei_url, W_u, b_u, W_p, b_p, W_l, b_l, Wl_pc0, bl_pc0, Wr_pc0, Wl_url0, bl_url0, Wr_url0, Wl_pc1, bl_pc1, Wr_pc1, Wl_url1, bl_url1, Wr_url1, W_agg, b_agg, W_c1, b_c1, W_c2, b_c2):
    raise NotImplementedError("write your pallas kernel here")


# speedup vs baseline: 11.2043x; 11.2043x over previous
import functools

import jax
import jax.numpy as jnp
from jax import lax
from jax.experimental import pallas as pl
from jax.experimental.pallas import tpu as pltpu
from jax.experimental.pallas import tpu_sc as plsc

F32 = jnp.float32
I32 = jnp.int32

NU = 50000
NS = 10000
E = 800000
NTILE = 16
CPT1 = 49
EPT = NTILE * CPT1 * 1024
PAD = EPT - E
RPT = CPT1 * 8

_MESH = plsc.VectorSubcoreMesh(core_axis_name="c", subcore_axis_name="s",
                               num_cores=2, num_subcores=16)
_SC_PARAMS = pltpu.CompilerParams(use_tc_tiling_on_sc=False)


def _seg_body(xu8, src1, dst1, z1, out, gi, si, rows, acc, sem):
    c = lax.axis_index("c")
    s = lax.axis_index("s")
    pltpu.sync_copy(z1.at[pl.ds(s * 640, 640)], acc.at[pl.ds(s * 640, 640)])
    plsc.subcore_barrier()

    def chunk(ct, carry):
        r0 = c * (NTILE * RPT) + s * RPT + ct * 8
        pltpu.sync_copy(src1.at[pl.ds(r0, 8)], gi)
        pltpu.sync_copy(dst1.at[pl.ds(r0, 8)], si)
        ds = [pltpu.async_copy(xu8.at[gi.at[j]], rows.at[j], sem)
              for j in range(8)]
        for d in ds:
            d.wait()
        for j in range(8):
            pltpu.sync_copy(rows.at[j], acc.at[si.at[j]], add=True)
        return carry

    lax.fori_loop(0, CPT1, chunk, 0)
    plsc.subcore_barrier()
    pltpu.sync_copy(acc.at[pl.ds(s * 640, 640)],
                    out.at[pl.ds(c * 10240 + s * 640, 640)])


_seg_kernel = pl.kernel(
    _seg_body,
    out_type=jax.ShapeDtypeStruct((2 * 10240, 8), F32),
    mesh=_MESH,
    scratch_types=[
        pltpu.VMEM((8, 128), I32),
        pltpu.VMEM((8, 128), I32),
        pltpu.VMEM((8, 128, 8), F32),
        pltpu.VMEM_SHARED((10240, 8), F32),
        pltpu.SemaphoreType.DMA,
    ],
    compiler_params=_SC_PARAMS,
)


def _ctx_body(tab, srcA, dstA, z3, out, gi, si, rows, acc, sem):
    c = lax.axis_index("c")
    s = lax.axis_index("s")
    pltpu.sync_copy(z3.at[pl.ds(s * 3128, 3128)], acc.at[pl.ds(s * 3128, 3128)])
    plsc.subcore_barrier()

    def chunk(ct, carry):
        r0 = s * (2 * RPT) + ct * 4
        pltpu.sync_copy(srcA.at[pl.ds(r0, 4)], si)
        pltpu.sync_copy(dstA.at[pl.ds(c * (NTILE * 2 * RPT) + r0, 4)], gi)
        ds = [pltpu.async_copy(tab.at[gi.at[j]], rows.at[j], sem)
              for j in range(4)]
        for d in ds:
            d.wait()
        for j in range(4):
            pltpu.sync_copy(rows.at[j], acc.at[si.at[j]], add=True)
        return carry

    lax.fori_loop(0, 4 * CPT1, chunk, 0)
    plsc.subcore_barrier()
    pltpu.sync_copy(acc.at[pl.ds(s * 3128, 3128)],
                    out.at[pl.ds(c * 50048 + s * 3128, 3128)])


_ctx_kernel = pl.kernel(
    _ctx_body,
    out_type=jax.ShapeDtypeStruct((2 * 50048, 32), F32),
    mesh=_MESH,
    scratch_types=[
        pltpu.VMEM((4, 128), I32),
        pltpu.VMEM((4, 128), I32),
        pltpu.VMEM((4, 128, 32), F32),
        pltpu.VMEM_SHARED((50048, 32), F32),
        pltpu.SemaphoreType.DMA,
    ],
    compiler_params=_SC_PARAMS,
)


def _node_body(Spc, Surl, xp8, xl8, Wu8, bu, Wp8, bp, Wl8, bl,
               Wlp0, blp0, Wrp0, Wlp1, blp1, Wrp1,
               Wll0, bll0, Wrl0, Wll1, bll1, Wrl1,
               Wap, Wal, T):
    def node(S, x8, Win8, bin_, Wl0, bl0, Wr0, Wl1, bl1, Wr1, Wa):
        cnt = S[:, 6:7]
        cl = jnp.maximum(cnt, 1.0)
        m = (jnp.dot(S, Wu8[...], preferred_element_type=F32)
             + cnt * bu[...]) / cl
        h_in = jnp.dot(x8, Win8, preferred_element_type=F32) + bin_
        h0 = jnp.maximum(
            jnp.dot(m, Wl0, preferred_element_type=F32) + bl0
            + jnp.dot(h_in, Wr0, preferred_element_type=F32), 0.0)
        h1 = jnp.maximum(
            jnp.dot(m, Wl1, preferred_element_type=F32) + bl1
            + jnp.dot(h0, Wr1, preferred_element_type=F32) + h0, 0.0)
        return jnp.dot(h1, Wa, preferred_element_type=F32)

    pt = node(Spc[...], xp8[...], Wp8[...], bp[...], Wlp0[...], blp0[...],
              Wrp0[...], Wlp1[...], blp1[...], Wrp1[...], Wap[...])
    lt = node(Surl[...], xl8[...], Wl8[...], bl[...], Wll0[...], bll0[...],
              Wrl0[...], Wll1[...], bll1[...], Wrl1[...], Wal[...])
    T[0:NS, :] = pt[:, :32]
    T[NS:2 * NS, :] = pt[:, 32:]
    T[2 * NS:3 * NS, :] = lt[:, :32]
    T[3 * NS:4 * NS, :] = lt[:, 32:]


def _fin_body(x8, a0, a1, Wu8, bu, Wau, bagg, Wc1, bc1, Wc2, bc2, o):
    u = jnp.dot(x8[...], Wu8[...], preferred_element_type=F32) + bu[...]
    enr = (jnp.dot(u, Wau[...], preferred_element_type=F32) + bagg[...]
           + jnp.concatenate([a0[...], a1[...]], axis=1))
    h = jnp.maximum(jnp.dot(enr, Wc1[...], preferred_element_type=F32)
                    + bc1[...], 0.0)
    o[...] = jnp.dot(h, Wc2[...], preferred_element_type=F32) + bc2[...]


def _pad8(x):
    n, k = x.shape
    return jnp.concatenate([x, jnp.zeros((n, 8 - k), F32)], axis=1)


def _w8(w):
    k, h = w.shape
    return jnp.concatenate([w, jnp.zeros((8 - k, h), F32)], axis=0)


def kernel(x_user, x_pc, x_url, ei_pc, ei_url,
           W_u, b_u, W_p, b_p, W_l, b_l,
           Wl_pc0, bl_pc0, Wr_pc0, Wl_url0, bl_url0, Wr_url0,
           Wl_pc1, bl_pc1, Wr_pc1, Wl_url1, bl_url1, Wr_url1,
           W_agg, b_agg, W_c1, b_c1, W_c2, b_c2):
    f = jnp.arange(PAD, dtype=I32)

    def tile_rows(a, fill):
        return jnp.concatenate([a, fill]).reshape(NTILE, RPT, 128)

    src1 = jnp.concatenate([
        tile_rows(ei_pc[0], f % NU), tile_rows(ei_url[0], f % NU)
    ]).reshape(2 * NTILE * RPT, 128)
    dst1 = jnp.concatenate([
        tile_rows(ei_pc[1], NS + (f % 16)), tile_rows(ei_url[1], NS + (f % 16))
    ]).reshape(2 * NTILE * RPT, 128)

    srcA = jnp.concatenate([
        tile_rows(ei_pc[0], NU + (f % 16)), tile_rows(ei_url[0], NU + (f % 16))
    ], axis=1).reshape(NTILE * 2 * RPT, 128)
    dbase = jnp.concatenate([
        tile_rows(ei_pc[1], f % NS), tile_rows(ei_url[1], f % NS) + 2 * NS
    ], axis=1)
    dstA = jnp.stack([dbase, dbase + NS]).reshape(2 * NTILE * 2 * RPT, 128)

    xu8 = jnp.concatenate(
        [x_user, jnp.ones((NU, 1), F32), jnp.zeros((NU, 1), F32)], axis=1)

    S = _seg_kernel(xu8, src1, dst1, jnp.zeros((10240, 8), F32))
    S_pc, S_url = S[:NS], S[10240:10240 + NS]

    r1 = lambda b: b.reshape(1, -1)
    T = pl.pallas_call(
        _node_body,
        out_shape=jax.ShapeDtypeStruct((4 * NS, 32), F32),
    )(S_pc, S_url, _pad8(x_pc), _pad8(x_url),
      _w8(W_u), r1(b_u), _w8(W_p), r1(b_p), _w8(W_l), r1(b_l),
      Wl_pc0, r1(bl_pc0), Wr_pc0, Wl_pc1, r1(bl_pc1), Wr_pc1,
      Wl_url0, r1(bl_url0), Wr_url0, Wl_url1, r1(bl_url1), Wr_url1,
      W_agg[64:128], W_agg[128:192])

    acc = _ctx_kernel(T, srcA, dstA, jnp.zeros((50048, 32), F32))
    a0, a1 = acc[:NU], acc[50048:50048 + NU]

    BU = 2000
    grid = NU // BU
    full = lambda shape: pl.BlockSpec(shape, lambda i: (0, 0))
    out = pl.pallas_call(
        _fin_body,
        grid=(grid,),
        in_specs=[
            pl.BlockSpec((BU, 8), lambda i: (i, 0)),
            pl.BlockSpec((BU, 32), lambda i: (i, 0)),
            pl.BlockSpec((BU, 32), lambda i: (i, 0)),
            full((8, 64)), full((1, 64)), full((64, 64)), full((1, 64)),
            full((64, 32)), full((1, 32)), full((32, 2)), full((1, 2)),
        ],
        out_specs=pl.BlockSpec((BU, 2), lambda i: (i, 0)),
        out_shape=jax.ShapeDtypeStruct((NU, 2), F32),
    )(_pad8(x_user), a0, a1, _w8(W_u), r1(b_u), W_agg[:64], r1(b_agg),
      W_c1, r1(b_c1), W_c2, r1(b_c2))
    return out


# speedup vs baseline: 12.3480x; 1.1021x over previous
import functools

import jax
import jax.numpy as jnp
from jax import lax
from jax.experimental import pallas as pl
from jax.experimental.pallas import tpu as pltpu
from jax.experimental.pallas import tpu_sc as plsc

F32 = jnp.float32
I32 = jnp.int32

NU = 50000
NS = 10000
E = 800000
NTILE = 16
CPTT = 131
RPT = CPTT * 3
EPT = NTILE * RPT * 128
PAD = EPT - E

_MESH = plsc.VectorSubcoreMesh(core_axis_name="c", subcore_axis_name="s",
                               num_cores=2, num_subcores=16)
_SC_PARAMS = pltpu.CompilerParams(use_tc_tiling_on_sc=False)


def _fire(tab, sidx, gidx, r, si, gi, rws, sem, off):
    pltpu.sync_copy(sidx.at[pl.ds(r, 3)], si)
    pltpu.sync_copy(gidx.at[pl.ds(r, 3)], gi)
    if off is not None:
        for j in range(3):
            for i in range(8):
                v = gi[j, pl.ds(i * 16, 16)]
                gi[j, pl.ds(i * 16, 16)] = v + off
    for j in range(3):
        pltpu.async_copy(tab.at[gi.at[j]], rws.at[j], sem)


def _drain_scatter(tab, acc, si, gi, rws, sem):
    for j in range(3):
        pltpu.make_async_copy(tab.at[gi.at[j]], rws.at[j], sem).wait()
    for j in range(3):
        pltpu.sync_copy(rws.at[j], acc.at[si.at[j]], add=True)


def _seg_body(xu8, srcA, dstA, z1, out,
              si0, gi0, r0b, si1, gi1, r1b, acc, semA, semB):
    c = lax.axis_index("c")
    s = lax.axis_index("s")
    pltpu.sync_copy(z1.at[pl.ds(s * 1876, 1876)],
                    acc.at[pl.ds(s * 1876, 1876)])
    plsc.subcore_barrier()
    base = s * (2 * RPT) + c * RPT

    _fire(xu8, dstA, srcA, base, si0, gi0, r0b, semA, None)

    def step(t, carry):
        @pl.when(t % 2 == 0)
        def _():
            @pl.when(t + 1 < CPTT)
            def _():
                _fire(xu8, dstA, srcA, base + (t + 1) * 3,
                      si1, gi1, r1b, semB, None)
            _drain_scatter(xu8, acc, si0, gi0, r0b, semA)

        @pl.when(t % 2 == 1)
        def _():
            @pl.when(t + 1 < CPTT)
            def _():
                _fire(xu8, dstA, srcA, base + (t + 1) * 3,
                      si0, gi0, r0b, semA, None)
            _drain_scatter(xu8, acc, si1, gi1, r1b, semB)

        return carry

    lax.fori_loop(0, CPTT, step, 0)
    plsc.subcore_barrier()
    pltpu.sync_copy(acc.at[pl.ds(c * 2 * NS + s * 625, 625)],
                    out.at[pl.ds(c * NS + s * 625, 625)])


_seg_kernel = pl.kernel(
    _seg_body,
    out_type=jax.ShapeDtypeStruct((2 * NS, 8), F32),
    mesh=_MESH,
    scratch_types=[
        pltpu.VMEM((3, 128), I32),
        pltpu.VMEM((3, 128), I32),
        pltpu.VMEM((3, 128, 8), F32),
        pltpu.VMEM((3, 128), I32),
        pltpu.VMEM((3, 128), I32),
        pltpu.VMEM((3, 128, 8), F32),
        pltpu.VMEM_SHARED((30016, 8), F32),
        pltpu.SemaphoreType.DMA,
        pltpu.SemaphoreType.DMA,
    ],
    compiler_params=_SC_PARAMS,
)


def _ctx_body(tab, srcA, dstA, z3, out,
              si0, gi0, r0b, si1, gi1, r1b, acc, semA, semB):
    c = lax.axis_index("c")
    s = lax.axis_index("s")
    off = c * NS
    pltpu.sync_copy(z3.at[pl.ds(s * 3128, 3128)],
                    acc.at[pl.ds(s * 3128, 3128)])
    plsc.subcore_barrier()
    base = s * (2 * RPT)
    NCH = 2 * CPTT

    _fire(tab, srcA, dstA, base, si0, gi0, r0b, semA, off)

    def step(t, carry):
        @pl.when(t % 2 == 0)
        def _():
            @pl.when(t + 1 < NCH)
            def _():
                _fire(tab, srcA, dstA, base + (t + 1) * 3,
                      si1, gi1, r1b, semB, off)
            _drain_scatter(tab, acc, si0, gi0, r0b, semA)

        @pl.when(t % 2 == 1)
        def _():
            @pl.when(t + 1 < NCH)
            def _():
                _fire(tab, srcA, dstA, base + (t + 1) * 3,
                      si0, gi0, r0b, semA, off)
            _drain_scatter(tab, acc, si1, gi1, r1b, semB)

        return carry

    lax.fori_loop(0, NCH, step, 0)
    plsc.subcore_barrier()
    pltpu.sync_copy(acc.at[pl.ds(s * 3128, 3128)],
                    out.at[pl.ds(c * 50048 + s * 3128, 3128)])


_ctx_kernel = pl.kernel(
    _ctx_body,
    out_type=jax.ShapeDtypeStruct((2 * 50048, 32), F32),
    mesh=_MESH,
    scratch_types=[
        pltpu.VMEM((3, 128), I32),
        pltpu.VMEM((3, 128), I32),
        pltpu.VMEM((3, 128, 32), F32),
        pltpu.VMEM((3, 128), I32),
        pltpu.VMEM((3, 128), I32),
        pltpu.VMEM((3, 128, 32), F32),
        pltpu.VMEM_SHARED((50048, 32), F32),
        pltpu.SemaphoreType.DMA,
        pltpu.SemaphoreType.DMA,
    ],
    compiler_params=_SC_PARAMS,
)


def _node_body(Spc, Surl, xp8, xl8, Wu8, bu, Wp8, bp, Wl8, bl,
               Wlp0, blp0, Wrp0, Wlp1, blp1, Wrp1,
               Wll0, bll0, Wrl0, Wll1, bll1, Wrl1,
               Wap, Wal, T):
    def node(S, x8, Win8, bin_, Wl0, bl0, Wr0, Wl1, bl1, Wr1, Wa):
        cnt = S[:, 6:7]
        cl = jnp.maximum(cnt, 1.0)
        m = (jnp.dot(S, Wu8[...], preferred_element_type=F32)
             + cnt * bu[...]) / cl
        h_in = jnp.dot(x8, Win8, preferred_element_type=F32) + bin_
        h0 = jnp.maximum(
            jnp.dot(m, Wl0, preferred_element_type=F32) + bl0
            + jnp.dot(h_in, Wr0, preferred_element_type=F32), 0.0)
        h1 = jnp.maximum(
            jnp.dot(m, Wl1, preferred_element_type=F32) + bl1
            + jnp.dot(h0, Wr1, preferred_element_type=F32) + h0, 0.0)
        return jnp.dot(h1, Wa, preferred_element_type=F32)

    pt = node(Spc[...], xp8[...], Wp8[...], bp[...], Wlp0[...], blp0[...],
              Wrp0[...], Wlp1[...], blp1[...], Wrp1[...], Wap[...])
    lt = node(Surl[...], xl8[...], Wl8[...], bl[...], Wll0[...], bll0[...],
              Wrl0[...], Wll1[...], bll1[...], Wrl1[...], Wal[...])
    T[0:NS, :] = pt[:, :32]
    T[NS:2 * NS, :] = pt[:, 32:]
    T[2 * NS:3 * NS, :] = lt[:, :32]
    T[3 * NS:4 * NS, :] = lt[:, 32:]


def _fin_body(x8, a0, a1, Wu8, bu, Wau, bagg, Wc1, bc1, Wc2, bc2, o):
    u = jnp.dot(x8[...], Wu8[...], preferred_element_type=F32) + bu[...]
    enr = (jnp.dot(u, Wau[...], preferred_element_type=F32) + bagg[...]
           + jnp.concatenate([a0[...], a1[...]], axis=1))
    h = jnp.maximum(jnp.dot(enr, Wc1[...], preferred_element_type=F32)
                    + bc1[...], 0.0)
    o[...] = jnp.dot(h, Wc2[...], preferred_element_type=F32) + bc2[...]


def _pad8(x):
    n, k = x.shape
    return jnp.concatenate([x, jnp.zeros((n, 8 - k), F32)], axis=1)


def _w8(w):
    k, h = w.shape
    return jnp.concatenate([w, jnp.zeros((8 - k, h), F32)], axis=0)


def kernel(x_user, x_pc, x_url, ei_pc, ei_url,
           W_u, b_u, W_p, b_p, W_l, b_l,
           Wl_pc0, bl_pc0, Wr_pc0, Wl_url0, bl_url0, Wr_url0,
           Wl_pc1, bl_pc1, Wr_pc1, Wl_url1, bl_url1, Wr_url1,
           W_agg, b_agg, W_c1, b_c1, W_c2, b_c2):
    f = jnp.arange(PAD, dtype=I32)

    def tile_rows(a, fill):
        return jnp.concatenate([a, fill]).reshape(NTILE, RPT, 128)

    srcA = jnp.concatenate([
        tile_rows(ei_pc[0], NU + (f % 16)),
        tile_rows(ei_url[0], NU + (f % 16)),
    ], axis=1).reshape(NTILE * 2 * RPT, 128)
    dstA = jnp.concatenate([
        tile_rows(ei_pc[1], f % NS),
        tile_rows(ei_url[1], f % NS) + 2 * NS,
    ], axis=1).reshape(NTILE * 2 * RPT, 128)

    xu8 = jnp.concatenate([
        jnp.concatenate(
            [x_user, jnp.ones((NU, 1), F32), jnp.zeros((NU, 1), F32)],
            axis=1),
        jnp.zeros((16, 8), F32),
    ])

    S = _seg_kernel(xu8, srcA, dstA, jnp.zeros((30016, 8), F32))
    S_pc, S_url = S[:NS], S[NS:2 * NS]

    r1 = lambda b: b.reshape(1, -1)
    T = pl.pallas_call(
        _node_body,
        out_shape=jax.ShapeDtypeStruct((4 * NS, 32), F32),
    )(S_pc, S_url, _pad8(x_pc), _pad8(x_url),
      _w8(W_u), r1(b_u), _w8(W_p), r1(b_p), _w8(W_l), r1(b_l),
      Wl_pc0, r1(bl_pc0), Wr_pc0, Wl_pc1, r1(bl_pc1), Wr_pc1,
      Wl_url0, r1(bl_url0), Wr_url0, Wl_url1, r1(bl_url1), Wr_url1,
      W_agg[64:128], W_agg[128:192])

    acc = _ctx_kernel(T, srcA, dstA, jnp.zeros((50048, 32), F32))
    a0, a1 = acc[:NU], acc[50048:50048 + NU]

    BU = 2000
    grid = NU // BU
    full = lambda shape: pl.BlockSpec(shape, lambda i: (0, 0))
    out = pl.pallas_call(
        _fin_body,
        grid=(grid,),
        in_specs=[
            pl.BlockSpec((BU, 8), lambda i: (i, 0)),
            pl.BlockSpec((BU, 32), lambda i: (i, 0)),
            pl.BlockSpec((BU, 32), lambda i: (i, 0)),
            full((8, 64)), full((1, 64)), full((64, 64)), full((1, 64)),
            full((64, 32)), full((1, 32)), full((32, 2)), full((1, 2)),
        ],
        out_specs=pl.BlockSpec((BU, 2), lambda i: (i, 0)),
        out_shape=jax.ShapeDtypeStruct((NU, 2), F32),
    )(_pad8(x_user), a0, a1, _w8(W_u), r1(b_u), W_agg[:64], r1(b_agg),
      W_c1, r1(b_c1), W_c2, r1(b_c2))
    return out


# speedup vs baseline: 15.6699x; 1.2690x over previous
import functools

import jax
import jax.numpy as jnp
from jax import lax
from jax.experimental import pallas as pl
from jax.experimental.pallas import tpu as pltpu
from jax.experimental.pallas import tpu_sc as plsc

F32 = jnp.float32
I32 = jnp.int32

NU = 50000
NS = 10000
E = 800000
NTILE = 16
CPTT = 132
RPT = CPTT * 3
EPT = NTILE * RPT * 128
PAD = EPT - E

_MESH = plsc.VectorSubcoreMesh(core_axis_name="c", subcore_axis_name="s",
                               num_cores=2, num_subcores=16)
_SC_PARAMS = pltpu.CompilerParams(use_tc_tiling_on_sc=False)


def _apply_off(gi, off):
    if off is None:
        return
    for j in range(6):
        for i in range(8):
            v = gi[j, pl.ds(i * 16, 16)]
            gi[j, pl.ds(i * 16, 16)] = v + off


def _edge_loop(tab, acc, sidx, gidx, base, ngroups, off,
               siA, giA, siB, giB, r0b, r1b, semA, semB, semI):
    def idx_async(si, gi, g):
        pltpu.async_copy(sidx.at[pl.ds(base + g * 6, 6)], si, semI)
        pltpu.async_copy(gidx.at[pl.ds(base + g * 6, 6)], gi, semI)

    def idx_wait(si, gi, g):
        pltpu.make_async_copy(sidx.at[pl.ds(base + g * 6, 6)], si, semI).wait()
        pltpu.make_async_copy(gidx.at[pl.ds(base + g * 6, 6)], gi, semI).wait()
        _apply_off(gi, off)

    def gathers(gi, j0, rws, sem):
        for j in range(3):
            pltpu.async_copy(tab.at[gi.at[j0 + j]], rws.at[j], sem)

    def drain_scat(gi, j0, si, rws, sem):
        for j in range(3):
            pltpu.make_async_copy(tab.at[gi.at[j0 + j]], rws.at[j], sem).wait()
        for j in range(3):
            pltpu.sync_copy(rws.at[j], acc.at[si.at[j0 + j]], add=True)

    pltpu.sync_copy(sidx.at[pl.ds(base, 6)], siA)
    pltpu.sync_copy(gidx.at[pl.ds(base, 6)], giA)
    _apply_off(giA, off)
    gathers(giA, 0, r0b, semA)

    def group(g, carry):
        def do(si_c, gi_c, si_n, gi_n):
            @pl.when(g + 1 < ngroups)
            def _():
                idx_async(si_n, gi_n, g + 1)
            gathers(gi_c, 3, r1b, semB)
            drain_scat(gi_c, 0, si_c, r0b, semA)

            @pl.when(g + 1 < ngroups)
            def _():
                idx_wait(si_n, gi_n, g + 1)
                gathers(gi_n, 0, r0b, semA)

            drain_scat(gi_c, 3, si_c, r1b, semB)

        @pl.when(g % 2 == 0)
        def _():
            do(siA, giA, siB, giB)

        @pl.when(g % 2 == 1)
        def _():
            do(siB, giB, siA, giA)

        return carry

    lax.fori_loop(0, ngroups, group, 0)


def _seg_body(xu8, srcA, dstA, z1, out,
              siA, giA, siB, giB, r0b, r1b, acc, semA, semB, semI):
    c = lax.axis_index("c")
    s = lax.axis_index("s")
    pltpu.sync_copy(z1.at[pl.ds(s * 1876, 1876)],
                    acc.at[pl.ds(s * 1876, 1876)])
    plsc.subcore_barrier()
    base = s * (2 * RPT) + c * RPT
    _edge_loop(xu8, acc, dstA, srcA, base, CPTT // 2, None,
               siA, giA, siB, giB, r0b, r1b, semA, semB, semI)
    plsc.subcore_barrier()
    pltpu.sync_copy(acc.at[pl.ds(c * 2 * NS + s * 625, 625)],
                    out.at[pl.ds(c * NS + s * 625, 625)])


_seg_kernel = pl.kernel(
    _seg_body,
    out_type=jax.ShapeDtypeStruct((2 * NS, 8), F32),
    mesh=_MESH,
    scratch_types=[
        pltpu.VMEM((6, 128), I32),
        pltpu.VMEM((6, 128), I32),
        pltpu.VMEM((6, 128), I32),
        pltpu.VMEM((6, 128), I32),
        pltpu.VMEM((3, 128, 8), F32),
        pltpu.VMEM((3, 128, 8), F32),
        pltpu.VMEM_SHARED((30016, 8), F32),
        pltpu.SemaphoreType.DMA,
        pltpu.SemaphoreType.DMA,
        pltpu.SemaphoreType.DMA,
    ],
    compiler_params=_SC_PARAMS,
)


def _ctx_body(tab, srcA, dstA, z3, out,
              siA, giA, siB, giB, r0b, r1b, acc, semA, semB, semI):
    c = lax.axis_index("c")
    s = lax.axis_index("s")
    pltpu.sync_copy(z3.at[pl.ds(s * 3128, 3128)],
                    acc.at[pl.ds(s * 3128, 3128)])
    plsc.subcore_barrier()
    base = s * (2 * RPT)
    _edge_loop(tab, acc, srcA, dstA, base, CPTT, c * NS,
               siA, giA, siB, giB, r0b, r1b, semA, semB, semI)
    plsc.subcore_barrier()
    pltpu.sync_copy(acc.at[pl.ds(s * 3128, 3128)],
                    out.at[pl.ds(c * 50048 + s * 3128, 3128)])


_ctx_kernel = pl.kernel(
    _ctx_body,
    out_type=jax.ShapeDtypeStruct((2 * 50048, 32), F32),
    mesh=_MESH,
    scratch_types=[
        pltpu.VMEM((6, 128), I32),
        pltpu.VMEM((6, 128), I32),
        pltpu.VMEM((6, 128), I32),
        pltpu.VMEM((6, 128), I32),
        pltpu.VMEM((3, 128, 32), F32),
        pltpu.VMEM((3, 128, 32), F32),
        pltpu.VMEM_SHARED((50048, 32), F32),
        pltpu.SemaphoreType.DMA,
        pltpu.SemaphoreType.DMA,
        pltpu.SemaphoreType.DMA,
    ],
    compiler_params=_SC_PARAMS,
)


def _node_body(Spc, Surl, xp8, xl8, Wu8, bu, Wp8, bp, Wl8, bl,
               Wlp0, blp0, Wrp0, Wlp1, blp1, Wrp1,
               Wll0, bll0, Wrl0, Wll1, bll1, Wrl1,
               Wap, Wal, T):
    def node(S, x8, Win8, bin_, Wl0, bl0, Wr0, Wl1, bl1, Wr1, Wa):
        cnt = S[:, 6:7]
        cl = jnp.maximum(cnt, 1.0)
        m = (jnp.dot(S, Wu8[...], preferred_element_type=F32)
             + cnt * bu[...]) / cl
        h_in = jnp.dot(x8, Win8, preferred_element_type=F32) + bin_
        h0 = jnp.maximum(
            jnp.dot(m, Wl0, preferred_element_type=F32) + bl0
            + jnp.dot(h_in, Wr0, preferred_element_type=F32), 0.0)
        h1 = jnp.maximum(
            jnp.dot(m, Wl1, preferred_element_type=F32) + bl1
            + jnp.dot(h0, Wr1, preferred_element_type=F32) + h0, 0.0)
        return jnp.dot(h1, Wa, preferred_element_type=F32)

    pt = node(Spc[...], xp8[...], Wp8[...], bp[...], Wlp0[...], blp0[...],
              Wrp0[...], Wlp1[...], blp1[...], Wrp1[...], Wap[...])
    lt = node(Surl[...], xl8[...], Wl8[...], bl[...], Wll0[...], bll0[...],
              Wrl0[...], Wll1[...], bll1[...], Wrl1[...], Wal[...])
    T[0:NS, :] = pt[:, :32]
    T[NS:2 * NS, :] = pt[:, 32:]
    T[2 * NS:3 * NS, :] = lt[:, :32]
    T[3 * NS:4 * NS, :] = lt[:, 32:]


def _fin_body(x8, a0, a1, Wu8, bu, Wau, bagg, Wc1, bc1, Wc2, bc2, o):
    u = jnp.dot(x8[...], Wu8[...], preferred_element_type=F32) + bu[...]
    enr = (jnp.dot(u, Wau[...], preferred_element_type=F32) + bagg[...]
           + jnp.concatenate([a0[...], a1[...]], axis=1))
    h = jnp.maximum(jnp.dot(enr, Wc1[...], preferred_element_type=F32)
                    + bc1[...], 0.0)
    o[...] = jnp.dot(h, Wc2[...], preferred_element_type=F32) + bc2[...]


def _pad8(x):
    n, k = x.shape
    return jnp.concatenate([x, jnp.zeros((n, 8 - k), F32)], axis=1)


def _w8(w):
    k, h = w.shape
    return jnp.concatenate([w, jnp.zeros((8 - k, h), F32)], axis=0)


def kernel(x_user, x_pc, x_url, ei_pc, ei_url,
           W_u, b_u, W_p, b_p, W_l, b_l,
           Wl_pc0, bl_pc0, Wr_pc0, Wl_url0, bl_url0, Wr_url0,
           Wl_pc1, bl_pc1, Wr_pc1, Wl_url1, bl_url1, Wr_url1,
           W_agg, b_agg, W_c1, b_c1, W_c2, b_c2):
    f = jnp.arange(PAD, dtype=I32)

    def tile_rows(a, fill):
        return jnp.concatenate([a, fill]).reshape(NTILE, RPT, 128)

    srcA = jnp.concatenate([
        tile_rows(ei_pc[0], NU + (f % 16)),
        tile_rows(ei_url[0], NU + (f % 16)),
    ], axis=1).reshape(NTILE * 2 * RPT, 128)
    dstA = jnp.concatenate([
        tile_rows(ei_pc[1], f % NS),
        tile_rows(ei_url[1], f % NS) + 2 * NS,
    ], axis=1).reshape(NTILE * 2 * RPT, 128)

    xu8 = jnp.concatenate([
        jnp.concatenate(
            [x_user, jnp.ones((NU, 1), F32), jnp.zeros((NU, 1), F32)],
            axis=1),
        jnp.zeros((16, 8), F32),
    ])

    S = _seg_kernel(xu8, srcA, dstA, jnp.zeros((30016, 8), F32))
    S_pc, S_url = S[:NS], S[NS:2 * NS]

    r1 = lambda b: b.reshape(1, -1)
    T = pl.pallas_call(
        _node_body,
        out_shape=jax.ShapeDtypeStruct((4 * NS, 32), F32),
    )(S_pc, S_url, _pad8(x_pc), _pad8(x_url),
      _w8(W_u), r1(b_u), _w8(W_p), r1(b_p), _w8(W_l), r1(b_l),
      Wl_pc0, r1(bl_pc0), Wr_pc0, Wl_pc1, r1(bl_pc1), Wr_pc1,
      Wl_url0, r1(bl_url0), Wr_url0, Wl_url1, r1(bl_url1), Wr_url1,
      W_agg[64:128], W_agg[128:192])

    acc = _ctx_kernel(T, srcA, dstA, jnp.zeros((50048, 32), F32))
    a0, a1 = acc[:NU], acc[50048:50048 + NU]

    BU = 2000
    grid = NU // BU
    full = lambda shape: pl.BlockSpec(shape, lambda i: (0, 0))
    out = pl.pallas_call(
        _fin_body,
        grid=(grid,),
        in_specs=[
            pl.BlockSpec((BU, 8), lambda i: (i, 0)),
            pl.BlockSpec((BU, 32), lambda i: (i, 0)),
            pl.BlockSpec((BU, 32), lambda i: (i, 0)),
            full((8, 64)), full((1, 64)), full((64, 64)), full((1, 64)),
            full((64, 32)), full((1, 32)), full((32, 2)), full((1, 2)),
        ],
        out_specs=pl.BlockSpec((BU, 2), lambda i: (i, 0)),
        out_shape=jax.ShapeDtypeStruct((NU, 2), F32),
    )(_pad8(x_user), a0, a1, _w8(W_u), r1(b_u), W_agg[:64], r1(b_agg),
      W_c1, r1(b_c1), W_c2, r1(b_c2))
    return out
